# baseline (speedup 1.0000x reference)
import jax
import jax.numpy as jnp
from jax import lax
from jax.experimental import pallas as pl
from jax.experimental.pallas import tpu as pltpu

N_FRAMES = 512
N_PIX = 128
N_CH = 3
BATCH = 64
PQ = N_PIX * N_CH


def _sig(x):
    return 1.0 / (1.0 + jnp.exp(-x))


def _rtvf_body(idx_ref, mmu_ref, mlv_ref, c_ref, h_ref, b_ref, v_ref,
               ns_ref, nh_ref, out_ref):
    mu = mmu_ref[0]
    lv = mlv_ref[0]
    c = c_ref[0, 0, 0]
    hrow = h_ref[0]
    s = b_ref[...] + c * v_ref[...]

    a = _sig(-mu)
    e = jnp.exp(lv)

    x = jnp.concatenate([a, e, ns_ref[0], nh_ref[0]], axis=0)
    col = lax.broadcasted_iota(jnp.int32, (N_PIX, PQ), 1) // N_CH
    row = lax.broadcasted_iota(jnp.int32, (N_PIX, PQ), 0)
    r = (col == row).astype(jnp.bfloat16)
    xh = x.astype(jnp.bfloat16)
    xl = (x - xh.astype(jnp.float32)).astype(jnp.bfloat16)
    y = (jnp.dot(xh, r, preferred_element_type=jnp.float32)
         + jnp.dot(xl, r, preferred_element_type=jnp.float32))

    a3 = y[0:N_PIX]
    e3 = y[N_PIX:2 * N_PIX]
    ns3 = y[2 * N_PIX:3 * N_PIX]
    nh3 = y[3 * N_PIX:4 * N_PIX]

    t = a3 * s + (1.0 - a3) * hrow + e3 * (s * s * ns3 + hrow * hrow * nh3)
    out_ref[0] = _sig(t)


def kernel(index, img, B, V, C, Mmu, Mlv, H, noise_S, noise_H):
    del img
    idx = index.astype(jnp.int32)
    h2 = H.reshape(N_FRAMES, N_PIX, PQ)
    b2 = B.reshape(N_PIX, PQ)
    v2 = V.reshape(N_PIX, PQ)
    c3 = C.reshape(N_FRAMES, 1, 1)
    ns = noise_S.reshape(BATCH, N_PIX, N_PIX)
    nh = noise_H.reshape(BATCH, N_PIX, N_PIX)

    grid_spec = pltpu.PrefetchScalarGridSpec(
        num_scalar_prefetch=1,
        grid=(BATCH,),
        in_specs=[
            pl.BlockSpec((1, N_PIX, N_PIX), lambda b, i: (i[b], 0, 0)),
            pl.BlockSpec((1, N_PIX, N_PIX), lambda b, i: (i[b], 0, 0)),
            pl.BlockSpec((1, 1, 1), lambda b, i: (i[b], 0, 0)),
            pl.BlockSpec((1, N_PIX, PQ), lambda b, i: (i[b], 0, 0)),
            pl.BlockSpec((N_PIX, PQ), lambda b, i: (0, 0)),
            pl.BlockSpec((N_PIX, PQ), lambda b, i: (0, 0)),
            pl.BlockSpec((1, N_PIX, N_PIX), lambda b, i: (b, 0, 0)),
            pl.BlockSpec((1, N_PIX, N_PIX), lambda b, i: (b, 0, 0)),
        ],
        out_specs=pl.BlockSpec((1, N_PIX, PQ), lambda b, i: (b, 0, 0)),
    )

    out = pl.pallas_call(
        _rtvf_body,
        grid_spec=grid_spec,
        out_shape=jax.ShapeDtypeStruct((BATCH, N_PIX, PQ), jnp.float32),
        compiler_params=pltpu.CompilerParams(
            dimension_semantics=("arbitrary",),
        ),
    )(idx, Mmu, Mlv, c3, h2, b2, v2, ns, nh)

    return out.reshape(BATCH, N_PIX, N_PIX, N_CH)


# speedup vs baseline: 5.5930x; 5.5930x over previous
import jax
import jax.numpy as jnp
from jax.experimental import pallas as pl
from jax.experimental.pallas import tpu as pltpu

N_FRAMES = 512
N_PIX = 128
N_CH = 3
BATCH = 64


def _sig(x):
    return 0.5 + 0.5 * jnp.tanh(0.5 * x)


def _rtvf_body(idx_ref, mmu_ref, mlv_ref, c_ref, h_ref, b_ref, v_ref,
               ns_ref, nh_ref, out_ref):
    b = pl.program_id(0)
    f = idx_ref[b]
    c = c_ref[0, f]

    mu = mmu_ref[0]
    th = jnp.tanh(0.5 * mu)
    a = 0.5 - 0.5 * th
    one_m_a = 0.5 + 0.5 * th
    e = jnp.exp(mlv_ref[0])
    ns = ns_ref[0]
    nh = nh_ref[0]

    for ch in range(N_CH):
        s = b_ref[ch] + c * v_ref[0, ch]
        hc = h_ref[0, ch]
        t = a * s + one_m_a * hc + e * (s * s * ns + hc * hc * nh)
        out_ref[0, ch] = _sig(t)


def kernel(index, img, B, V, C, Mmu, Mlv, H, noise_S, noise_H):
    del img
    idx = index.astype(jnp.int32)
    ht = jnp.transpose(H, (0, 3, 1, 2))
    vt = jnp.transpose(V, (0, 3, 1, 2))
    bt = jnp.transpose(B, (2, 0, 1))
    ct = jnp.transpose(C, (1, 0))
    ns = noise_S.reshape(BATCH, N_PIX, N_PIX)
    nh = noise_H.reshape(BATCH, N_PIX, N_PIX)

    grid_spec = pltpu.PrefetchScalarGridSpec(
        num_scalar_prefetch=1,
        grid=(BATCH,),
        in_specs=[
            pl.BlockSpec((1, N_PIX, N_PIX), lambda b, i: (i[b], 0, 0)),
            pl.BlockSpec((1, N_PIX, N_PIX), lambda b, i: (i[b], 0, 0)),
            pl.BlockSpec(memory_space=pltpu.SMEM),
            pl.BlockSpec((1, N_CH, N_PIX, N_PIX),
                         lambda b, i: (i[b], 0, 0, 0)),
            pl.BlockSpec((N_CH, N_PIX, N_PIX), lambda b, i: (0, 0, 0)),
            pl.BlockSpec((1, N_CH, N_PIX, N_PIX),
                         lambda b, i: (0, 0, 0, 0)),
            pl.BlockSpec((1, N_PIX, N_PIX), lambda b, i: (b, 0, 0)),
            pl.BlockSpec((1, N_PIX, N_PIX), lambda b, i: (b, 0, 0)),
        ],
        out_specs=pl.BlockSpec((1, N_CH, N_PIX, N_PIX),
                               lambda b, i: (b, 0, 0, 0)),
    )

    out = pl.pallas_call(
        _rtvf_body,
        grid_spec=grid_spec,
        out_shape=jax.ShapeDtypeStruct((BATCH, N_CH, N_PIX, N_PIX),
                                       jnp.float32),
        compiler_params=pltpu.CompilerParams(
            dimension_semantics=("arbitrary",),
        ),
    )(idx, Mmu, Mlv, ct, ht, bt, vt, ns, nh)

    return jnp.transpose(out, (0, 2, 3, 1))


# speedup vs baseline: 11.2634x; 2.0139x over previous
import jax
import jax.numpy as jnp
from jax.experimental import pallas as pl
from jax.experimental.pallas import tpu as pltpu

N_FRAMES = 512
N_PIX = 128
N_CH = 3
BATCH = 64
PER_STEP = 4
STEPS = BATCH // PER_STEP


def _sig(x):
    return 0.5 + 0.5 * jnp.tanh(0.5 * x)


def _rtvf_body(idx_ref, *refs):
    c_ref, b_ref, v_ref, ns_ref, nh_ref = refs[3 * PER_STEP:3 * PER_STEP + 5]
    out_ref = refs[-1]
    step = pl.program_id(0)

    for k in range(PER_STEP):
        mmu_ref, mlv_ref, h_ref = refs[3 * k:3 * k + 3]
        f = idx_ref[step * PER_STEP + k]
        c = c_ref[0, f]

        th = jnp.tanh(0.5 * mmu_ref[0])
        a = 0.5 - 0.5 * th
        one_m_a = 0.5 + 0.5 * th
        e = jnp.exp(mlv_ref[0])
        ns = ns_ref[k]
        nh = nh_ref[k]

        for ch in range(N_CH):
            s = b_ref[ch] + c * v_ref[0, ch]
            hc = h_ref[0, ch]
            t = a * s + one_m_a * hc + e * (s * s * ns + hc * hc * nh)
            out_ref[k, ch] = _sig(t)


def kernel(index, img, B, V, C, Mmu, Mlv, H, noise_S, noise_H):
    del img
    idx = index.astype(jnp.int32)
    ht = jnp.transpose(H, (0, 3, 1, 2))
    vt = jnp.transpose(V, (0, 3, 1, 2))
    bt = jnp.transpose(B, (2, 0, 1))
    ct = jnp.transpose(C, (1, 0))
    ns = noise_S.reshape(BATCH, N_PIX, N_PIX)
    nh = noise_H.reshape(BATCH, N_PIX, N_PIX)

    def gspec(k):
        return lambda b, i: (i[b * PER_STEP + k], 0, 0)

    def gspec4(k):
        return lambda b, i: (i[b * PER_STEP + k], 0, 0, 0)

    in_specs = []
    operands = []
    for k in range(PER_STEP):
        in_specs.append(pl.BlockSpec((1, N_PIX, N_PIX), gspec(k)))
        in_specs.append(pl.BlockSpec((1, N_PIX, N_PIX), gspec(k)))
        in_specs.append(pl.BlockSpec((1, N_CH, N_PIX, N_PIX), gspec4(k)))
        operands.extend([Mmu, Mlv, ht])
    in_specs.extend([
        pl.BlockSpec(memory_space=pltpu.SMEM),
        pl.BlockSpec((N_CH, N_PIX, N_PIX), lambda b, i: (0, 0, 0)),
        pl.BlockSpec((1, N_CH, N_PIX, N_PIX),
                     lambda b, i: (0, 0, 0, 0)),
        pl.BlockSpec((PER_STEP, N_PIX, N_PIX), lambda b, i: (b, 0, 0)),
        pl.BlockSpec((PER_STEP, N_PIX, N_PIX), lambda b, i: (b, 0, 0)),
    ])
    operands.extend([ct, bt, vt, ns, nh])

    grid_spec = pltpu.PrefetchScalarGridSpec(
        num_scalar_prefetch=1,
        grid=(STEPS,),
        in_specs=in_specs,
        out_specs=pl.BlockSpec((PER_STEP, N_CH, N_PIX, N_PIX),
                               lambda b, i: (b, 0, 0, 0)),
    )

    out = pl.pallas_call(
        _rtvf_body,
        grid_spec=grid_spec,
        out_shape=jax.ShapeDtypeStruct((BATCH, N_CH, N_PIX, N_PIX),
                                       jnp.float32),
        compiler_params=pltpu.CompilerParams(
            dimension_semantics=("arbitrary",),
        ),
    )(idx, *operands)

    return jnp.transpose(out, (0, 2, 3, 1))


# speedup vs baseline: 13.3570x; 1.1859x over previous
import jax
import jax.numpy as jnp
from jax.experimental import pallas as pl
from jax.experimental.pallas import tpu as pltpu

N_FRAMES = 512
N_PIX = 128
N_CH = 3
BATCH = 64
PER_STEP = 8
STEPS = BATCH // PER_STEP


def _sig(x):
    return 0.5 + 0.5 * jnp.tanh(0.5 * x)


def _rtvf_body(idx_ref, *refs):
    c_ref, b_ref, v_ref, ns_ref, nh_ref = refs[3 * PER_STEP:3 * PER_STEP + 5]
    out_ref = refs[-1]
    step = pl.program_id(0)

    for k in range(PER_STEP):
        mmu_ref, mlv_ref, h_ref = refs[3 * k:3 * k + 3]
        f = idx_ref[step * PER_STEP + k]
        c = c_ref[0, f]

        th = jnp.tanh(0.5 * mmu_ref[0])
        a = 0.5 - 0.5 * th
        one_m_a = 0.5 + 0.5 * th
        e = jnp.exp(mlv_ref[0])
        ns = ns_ref[k]
        nh = nh_ref[k]

        for ch in range(N_CH):
            s = b_ref[ch] + c * v_ref[0, ch]
            hc = h_ref[0, ch]
            t = a * s + one_m_a * hc + e * (s * s * ns + hc * hc * nh)
            out_ref[k, ch] = _sig(t)


def kernel(index, img, B, V, C, Mmu, Mlv, H, noise_S, noise_H):
    del img
    idx = index.astype(jnp.int32)
    ht = jnp.transpose(H, (0, 3, 1, 2))
    vt = jnp.transpose(V, (0, 3, 1, 2))
    bt = jnp.transpose(B, (2, 0, 1))
    ct = jnp.transpose(C, (1, 0))
    ns = noise_S.reshape(BATCH, N_PIX, N_PIX)
    nh = noise_H.reshape(BATCH, N_PIX, N_PIX)

    def gspec(k):
        return lambda b, i: (i[b * PER_STEP + k], 0, 0)

    def gspec4(k):
        return lambda b, i: (i[b * PER_STEP + k], 0, 0, 0)

    in_specs = []
    operands = []
    for k in range(PER_STEP):
        in_specs.append(pl.BlockSpec((1, N_PIX, N_PIX), gspec(k)))
        in_specs.append(pl.BlockSpec((1, N_PIX, N_PIX), gspec(k)))
        in_specs.append(pl.BlockSpec((1, N_CH, N_PIX, N_PIX), gspec4(k)))
        operands.extend([Mmu, Mlv, ht])
    in_specs.extend([
        pl.BlockSpec(memory_space=pltpu.SMEM),
        pl.BlockSpec((N_CH, N_PIX, N_PIX), lambda b, i: (0, 0, 0)),
        pl.BlockSpec((1, N_CH, N_PIX, N_PIX),
                     lambda b, i: (0, 0, 0, 0)),
        pl.BlockSpec((PER_STEP, N_PIX, N_PIX), lambda b, i: (b, 0, 0)),
        pl.BlockSpec((PER_STEP, N_PIX, N_PIX), lambda b, i: (b, 0, 0)),
    ])
    operands.extend([ct, bt, vt, ns, nh])

    grid_spec = pltpu.PrefetchScalarGridSpec(
        num_scalar_prefetch=1,
        grid=(STEPS,),
        in_specs=in_specs,
        out_specs=pl.BlockSpec((PER_STEP, N_CH, N_PIX, N_PIX),
                               lambda b, i: (b, 0, 0, 0)),
    )

    out = pl.pallas_call(
        _rtvf_body,
        grid_spec=grid_spec,
        out_shape=jax.ShapeDtypeStruct((BATCH, N_CH, N_PIX, N_PIX),
                                       jnp.float32),
        compiler_params=pltpu.CompilerParams(
            dimension_semantics=("arbitrary",),
        ),
    )(idx, *operands)

    return jnp.transpose(out, (0, 2, 3, 1))


# speedup vs baseline: 13.6989x; 1.0256x over previous
import jax
import jax.numpy as jnp
from jax.experimental import pallas as pl
from jax.experimental.pallas import tpu as pltpu

N_FRAMES = 512
N_PIX = 128
N_CH = 3
BATCH = 64
PER_STEP = 16
STEPS = BATCH // PER_STEP


def _sig(x):
    return 0.5 + 0.5 * jnp.tanh(0.5 * x)


def _rtvf_body(idx_ref, *refs):
    c_ref, b_ref, v_ref, ns_ref, nh_ref = refs[3 * PER_STEP:3 * PER_STEP + 5]
    out_ref = refs[-1]
    step = pl.program_id(0)

    for k in range(PER_STEP):
        mmu_ref, mlv_ref, h_ref = refs[3 * k:3 * k + 3]
        f = idx_ref[step * PER_STEP + k]
        c = c_ref[0, f]

        th = jnp.tanh(0.5 * mmu_ref[0])
        a = 0.5 - 0.5 * th
        one_m_a = 0.5 + 0.5 * th
        e = jnp.exp(mlv_ref[0])
        ns = ns_ref[k]
        nh = nh_ref[k]

        for ch in range(N_CH):
            s = b_ref[ch] + c * v_ref[0, ch]
            hc = h_ref[0, ch]
            t = a * s + one_m_a * hc + e * (s * s * ns + hc * hc * nh)
            out_ref[k, ch] = _sig(t)


def kernel(index, img, B, V, C, Mmu, Mlv, H, noise_S, noise_H):
    del img
    idx = index.astype(jnp.int32)
    ht = jnp.transpose(H, (0, 3, 1, 2))
    vt = jnp.transpose(V, (0, 3, 1, 2))
    bt = jnp.transpose(B, (2, 0, 1))
    ct = jnp.transpose(C, (1, 0))
    ns = noise_S.reshape(BATCH, N_PIX, N_PIX)
    nh = noise_H.reshape(BATCH, N_PIX, N_PIX)

    def gspec(k):
        return lambda b, i: (i[b * PER_STEP + k], 0, 0)

    def gspec4(k):
        return lambda b, i: (i[b * PER_STEP + k], 0, 0, 0)

    in_specs = []
    operands = []
    for k in range(PER_STEP):
        in_specs.append(pl.BlockSpec((1, N_PIX, N_PIX), gspec(k)))
        in_specs.append(pl.BlockSpec((1, N_PIX, N_PIX), gspec(k)))
        in_specs.append(pl.BlockSpec((1, N_CH, N_PIX, N_PIX), gspec4(k)))
        operands.extend([Mmu, Mlv, ht])
    in_specs.extend([
        pl.BlockSpec(memory_space=pltpu.SMEM),
        pl.BlockSpec((N_CH, N_PIX, N_PIX), lambda b, i: (0, 0, 0)),
        pl.BlockSpec((1, N_CH, N_PIX, N_PIX),
                     lambda b, i: (0, 0, 0, 0)),
        pl.BlockSpec((PER_STEP, N_PIX, N_PIX), lambda b, i: (b, 0, 0)),
        pl.BlockSpec((PER_STEP, N_PIX, N_PIX), lambda b, i: (b, 0, 0)),
    ])
    operands.extend([ct, bt, vt, ns, nh])

    grid_spec = pltpu.PrefetchScalarGridSpec(
        num_scalar_prefetch=1,
        grid=(STEPS,),
        in_specs=in_specs,
        out_specs=pl.BlockSpec((PER_STEP, N_CH, N_PIX, N_PIX),
                               lambda b, i: (b, 0, 0, 0)),
    )

    out = pl.pallas_call(
        _rtvf_body,
        grid_spec=grid_spec,
        out_shape=jax.ShapeDtypeStruct((BATCH, N_CH, N_PIX, N_PIX),
                                       jnp.float32),
        compiler_params=pltpu.CompilerParams(
            dimension_semantics=("arbitrary",),
        ),
    )(idx, *operands)

    return jnp.transpose(out, (0, 2, 3, 1))
